# baseline (speedup 1.0000x reference)
import functools

import jax
import jax.numpy as jnp
from jax import lax
from jax.experimental import pallas as pl
from jax.experimental.pallas import tpu as pltpu
from jax.experimental.pallas import tpu_sc as plsc

_NUM_USERS = 25000
_NUM_ITEMS = 25000
_D = 64
_HALF = 32
_N = _NUM_USERS + _NUM_ITEMS
_E = 800000
_NS = 16
_NPAD = 51200
_RPT = _NPAD // _NS
_WCH = 80
_NWCH = _RPT // _WCH
_NBUF = 5
_CHUNK = 80
_EPT = _E // _NS
_BLK = 25
_NBLK = _EPT // (_CHUNK * _BLK)
_NCHROWS = _E // _CHUNK


def _lgcn_body(x0, src3, dst2, out_sum, za, zb,
               acc, dacc, ones1, srcb, dstb, rows, dinv,
               gsems, ssems, isems):
    c = lax.axis_index("c")
    s = lax.axis_index("s")
    row0 = s * _RPT
    nbase = c * _NPAD + row0
    blk0_d = s * _NBLK
    blk0_s = c * (_NS * _NBLK) + blk0_d

    f1 = jnp.full((16,), 1.0, jnp.float32)
    f0 = jnp.zeros((16,), jnp.float32)
    wb0 = rows.at[0]
    wb1 = rows.at[1]

    def _clear_acc_slice():
        def zf(r, _):
            wb0[r, 0:16] = f0
            wb0[r, 16:32] = f0
            return 0
        lax.fori_loop(0, _WCH, zf, 0)
        def f(w, _):
            pltpu.sync_copy(wb0, acc.at[pl.ds(row0 + w * _WCH, _WCH)])
            return 0
        lax.fori_loop(0, _NWCH, f, 0)

    def _edge_pass(zsrc):
        deg = zsrc is None

        def load_idx(setk, b):
            ds_ = [pltpu.async_copy(dst2.at[blk0_d + b], dstb.at[setk],
                                    isems.at[0])]
            if not deg:
                ds_.append(pltpu.async_copy(src3.at[blk0_s + b], srcb.at[setk],
                                            isems.at[1]))
            return ds_

        def process(setk, b):
            sb = srcb.at[setk]
            db = dstb.at[setk]
            if deg:
                descs = [pltpu.async_copy(ones1, dacc.at[db.at[j]],
                                          ssems.at[j % _NBUF], add=True)
                         for j in range(_BLK)]
                for d in descs:
                    d.wait()
            else:
                def gather(j):
                    return pltpu.async_copy(zsrc.at[sb.at[j]],
                                            rows.at[j % _NBUF],
                                            gsems.at[j % _NBUF])
                def scatter(j):
                    return pltpu.async_copy(rows.at[j % _NBUF],
                                            acc.at[db.at[j]],
                                            ssems.at[j % _NBUF], add=True)
                gd = {j: gather(j) for j in range(_NBUF - 1)}
                sd = {}
                for j in range(_BLK):
                    if j + _NBUF - 1 < _BLK:
                        if j >= 1:
                            sd[j - 1].wait()
                        gd[j + _NBUF - 1] = gather(j + _NBUF - 1)
                    gd[j].wait()
                    sd[j] = scatter(j)
                for j in range(max(_BLK - _NBUF, 0), _BLK):
                    sd[j].wait()

        for d in load_idx(0, 0):
            d.wait()
        def pair(p, _):
            b0 = 2 * p
            d1 = load_idx(1, b0 + 1)
            process(0, b0)
            for d in d1:
                d.wait()
            d0 = load_idx(0, b0 + 2)
            process(1, b0 + 1)
            for d in d0:
                d.wait()
            return 0
        lax.fori_loop(0, _NBLK // 2, pair, 0)
        process(0, _NBLK - 1)

    def _dinv_phase():
        magic = jnp.full((16,), 0x5F3759DF, jnp.int32)
        one_i = jnp.full((16,), 1, jnp.int32)
        pltpu.sync_copy(dacc.at[pl.ds(row0, _RPT)], dinv)
        def gf(g, _):
            d = dinv[pl.ds(g * 16, 16)]
            ib = lax.bitcast_convert_type(d, jnp.int32)
            y = lax.bitcast_convert_type(
                magic - lax.shift_right_logical(ib, one_i), jnp.float32)
            for _i in range(4):
                y = y * (1.5 - 0.5 * d * y * y)
            y = jnp.where(d > 0.5, y, 0.0)
            dinv[pl.ds(g * 16, 16)] = y
            return 0
        lax.fori_loop(0, _RPT // 16, gf, 0)

    def _z0_phase():
        def wchunk(w, _):
            nb = nbase + w * _WCH
            pltpu.sync_copy(x0.at[pl.ds(nb, _WCH)], wb0)
            pltpu.sync_copy(wb0, out_sum.at[pl.ds(nb, _WCH)])
            def gf(g, _):
                dvec = dinv[pl.ds(w * _WCH + g * 16, 16)]
                for r16 in range(16):
                    r = g * 16 + r16
                    di = dvec[r16]
                    wb0[r, 0:16] = wb0[r, 0:16] * di
                    wb0[r, 16:32] = wb0[r, 16:32] * di
                return 0
            lax.fori_loop(0, _WCH // 16, gf, 0)
            pltpu.sync_copy(wb0, za.at[pl.ds(nb, _WCH)])
            return 0
        lax.fori_loop(0, _NWCH, wchunk, 0)

    def _writeback(last, zdst):
        def wchunk(w, _):
            nb = nbase + w * _WCH
            da = pltpu.async_copy(acc.at[pl.ds(row0 + w * _WCH, _WCH)], wb0,
                                  gsems.at[0])
            db = pltpu.async_copy(out_sum.at[pl.ds(nb, _WCH)], wb1,
                                  gsems.at[1])
            da.wait()
            db.wait()
            def gf(g, _):
                dvec = dinv[pl.ds(w * _WCH + g * 16, 16)]
                for r16 in range(16):
                    r = g * 16 + r16
                    di = dvec[r16]
                    s0 = wb1[r, 0:16] + wb0[r, 0:16] * di
                    s1 = wb1[r, 16:32] + wb0[r, 16:32] * di
                    if last:
                        wb1[r, 0:16] = s0 * 0.25
                        wb1[r, 16:32] = s1 * 0.25
                    else:
                        wb1[r, 0:16] = s0
                        wb1[r, 16:32] = s1
                        d2 = di * di
                        wb0[r, 0:16] = wb0[r, 0:16] * d2
                        wb0[r, 16:32] = wb0[r, 16:32] * d2
                return 0
            lax.fori_loop(0, _WCH // 16, gf, 0)
            pltpu.sync_copy(wb1, out_sum.at[pl.ds(nb, _WCH)])
            if not last:
                pltpu.sync_copy(wb0, zdst.at[pl.ds(nb, _WCH)])
            return 0
        lax.fori_loop(0, _NWCH, wchunk, 0)

    for k in range(_CHUNK // 16):
        ones1[pl.ds(k * 16, 16)] = f1
    def zf(g, _):
        dinv[pl.ds(g * 16, 16)] = f0
        return 0
    lax.fori_loop(0, _RPT // 16, zf, 0)
    pltpu.sync_copy(dinv, dacc.at[pl.ds(row0, _RPT)])
    plsc.subcore_barrier()
    with jax.named_scope("deg_pass"):
        _edge_pass(None)
    plsc.subcore_barrier()
    with jax.named_scope("dinv_z0"):
        _dinv_phase()
        _z0_phase()

    zsrc = za
    for l in range(3):
        with jax.named_scope(f"clear{l}"):
            _clear_acc_slice()
        plsc.subcore_barrier()
        with jax.named_scope(f"edges{l}"):
            _edge_pass(zsrc)
        plsc.subcore_barrier()
        zdst = zb if zsrc is za else za
        with jax.named_scope(f"wb{l}"):
            _writeback(last=(l == 2), zdst=zdst)
        zsrc = zdst


_lgcn = functools.partial(
    pl.kernel,
    out_type=(
        jax.ShapeDtypeStruct((2 * _NPAD, _HALF), jnp.float32),
        jax.ShapeDtypeStruct((2 * _NPAD, _HALF), jnp.float32),
        jax.ShapeDtypeStruct((2 * _NPAD, _HALF), jnp.float32),
    ),
    mesh=plsc.VectorSubcoreMesh(core_axis_name="c", subcore_axis_name="s"),
    compiler_params=pltpu.CompilerParams(use_tc_tiling_on_sc=False),
    scratch_types=[
        pltpu.VMEM_SHARED((_NPAD, _HALF), jnp.float32),
        pltpu.VMEM_SHARED((_NPAD,), jnp.float32),
        pltpu.VMEM((_CHUNK,), jnp.float32),
        pltpu.VMEM((2, _BLK, _CHUNK), jnp.int32),
        pltpu.VMEM((2, _BLK, _CHUNK), jnp.int32),
        pltpu.VMEM((_NBUF, _CHUNK, _HALF), jnp.float32),
        pltpu.VMEM((_RPT,), jnp.float32),
        pltpu.SemaphoreType.DMA((_NBUF,)),
        pltpu.SemaphoreType.DMA((_NBUF,)),
        pltpu.SemaphoreType.DMA((2,)),
    ],
)(_lgcn_body)


def kernel(user_table, item_table, edge_index):
    all_emb = jnp.concatenate([user_table, item_table], axis=0)
    x0 = jnp.pad(all_emb, ((0, _NPAD - _N), (0, 0)))
    x0 = x0.reshape(_NPAD, 2, _HALF).transpose(1, 0, 2).reshape(2 * _NPAD, _HALF)
    nblk_tot = _NCHROWS // _BLK
    src = edge_index[0].reshape(nblk_tot, _BLK, _CHUNK)
    src3 = jnp.concatenate([src, src + _NPAD], axis=0)
    dst2 = edge_index[1].reshape(nblk_tot, _BLK, _CHUNK)
    out_sum, _, _ = _lgcn(x0, src3, dst2)
    final = out_sum.reshape(2, _NPAD, _HALF).transpose(1, 0, 2)
    final = final.reshape(_NPAD, _D)[:_N]
    return final[:_NUM_USERS], final[_NUM_USERS:]


# speedup vs baseline: 1.2154x; 1.2154x over previous
import functools

import jax
import jax.numpy as jnp
from jax import lax
from jax.experimental import pallas as pl
from jax.experimental.pallas import tpu as pltpu
from jax.experimental.pallas import tpu_sc as plsc

_NUM_USERS = 25000
_NUM_ITEMS = 25000
_D = 64
_HALF = 32
_N = _NUM_USERS + _NUM_ITEMS
_E = 800000
_NS = 16
_NPAD = 51200
_RPT = _NPAD // _NS
_WCH = 80
_NWCH = _RPT // _WCH
_NBUF = 5
_CHUNK = 80
_EPT = _E // _NS
_BLK = 25
_NBLK = _EPT // (_CHUNK * _BLK)
_NCHROWS = _E // _CHUNK


def _lgcn_body(x0, src3, dst2, out_sum, za, zb,
               acc, dacc, ones1, srcb, dstb, rows, dinv,
               gsems, ssems, isems):
    c = lax.axis_index("c")
    s = lax.axis_index("s")
    row0 = s * _RPT
    zoff = c * _NPAD
    col0 = c * _HALF
    blk0 = s * _NBLK

    f1 = jnp.full((16,), 1.0, jnp.float32)
    f0 = jnp.zeros((16,), jnp.float32)

    def _clear_acc_slice():
        zbuf = rows.at[4]
        def zf(r, _):
            zbuf[r, 0:16] = f0
            zbuf[r, 16:32] = f0
            return 0
        lax.fori_loop(0, _WCH, zf, 0)
        def f(w, _):
            pltpu.sync_copy(zbuf, acc.at[pl.ds(row0 + w * _WCH, _WCH)])
            return 0
        lax.fori_loop(0, _NWCH, f, 0)

    def _edge_pass(zsrc):
        deg = zsrc is None
        ztab = None if deg else zsrc.at[pl.ds(zoff, _NPAD)]

        def load_idx(setk, b):
            ds_ = [pltpu.async_copy(dst2.at[blk0 + b], dstb.at[setk],
                                    isems.at[0])]
            if not deg:
                ds_.append(pltpu.async_copy(src3.at[blk0 + b], srcb.at[setk],
                                            isems.at[1]))
            return ds_

        def process(setk, b):
            sb = srcb.at[setk]
            db = dstb.at[setk]
            if deg:
                descs = [pltpu.async_copy(ones1, dacc.at[db.at[j]],
                                          ssems.at[j % _NBUF], add=True)
                         for j in range(_BLK)]
                for d in descs:
                    d.wait()
            else:
                def gather(j):
                    return pltpu.async_copy(ztab.at[sb.at[j]],
                                            rows.at[j % _NBUF],
                                            gsems.at[j % _NBUF])
                def scatter(j):
                    return pltpu.async_copy(rows.at[j % _NBUF],
                                            acc.at[db.at[j]],
                                            ssems.at[j % _NBUF], add=True)
                gd = {j: gather(j) for j in range(_NBUF - 1)}
                sd = {}
                for j in range(_BLK):
                    if j + _NBUF - 1 < _BLK:
                        if j >= 1:
                            sd[j - 1].wait()
                        gd[j + _NBUF - 1] = gather(j + _NBUF - 1)
                    gd[j].wait()
                    sd[j] = scatter(j)
                for j in range(max(_BLK - _NBUF, 0), _BLK):
                    sd[j].wait()

        for d in load_idx(0, 0):
            d.wait()
        def pair(p, _):
            b0 = 2 * p
            d1 = load_idx(1, b0 + 1)
            process(0, b0)
            for d in d1:
                d.wait()
            d0 = load_idx(0, b0 + 2)
            process(1, b0 + 1)
            for d in d0:
                d.wait()
            return 0
        lax.fori_loop(0, _NBLK // 2, pair, 0)
        process(0, _NBLK - 1)

    def _dinv_phase():
        magic = jnp.full((16,), 0x5F3759DF, jnp.int32)
        one_i = jnp.full((16,), 1, jnp.int32)
        pltpu.sync_copy(dacc.at[pl.ds(row0, _RPT)], dinv)
        def gf(g, _):
            d = dinv[pl.ds(g * 16, 16)]
            ib = lax.bitcast_convert_type(d, jnp.int32)
            y = lax.bitcast_convert_type(
                magic - lax.shift_right_logical(ib, one_i), jnp.float32)
            for _i in range(4):
                y = y * (1.5 - 0.5 * d * y * y)
            y = jnp.where(d > 0.5, y, 0.0)
            dinv[pl.ds(g * 16, 16)] = y
            return 0
        lax.fori_loop(0, _RPT // 16, gf, 0)

    def _z0_phase():
        def load(w, k):
            hb = row0 + w * _WCH
            return pltpu.async_copy(
                x0.at[pl.ds(hb, _WCH), pl.ds(col0, _HALF)], rows.at[k],
                gsems.at[k])
        def compute_store(w, k):
            hb = row0 + w * _WCH
            a = rows.at[k]
            z = rows.at[k + 1]
            def gf(g, _):
                dvec = dinv[pl.ds(w * _WCH + g * 16, 16)]
                for r16 in range(16):
                    r = g * 16 + r16
                    di = dvec[r16]
                    z[r, 0:16] = a[r, 0:16] * di
                    z[r, 16:32] = a[r, 16:32] * di
                return 0
            lax.fori_loop(0, _WCH // 16, gf, 0)
            return [
                pltpu.async_copy(
                    a, out_sum.at[pl.ds(hb, _WCH), pl.ds(col0, _HALF)],
                    ssems.at[k]),
                pltpu.async_copy(z, za.at[pl.ds(zoff + hb, _WCH)],
                                 ssems.at[k + 1]),
            ]
        def pairf(p, _):
            w0 = 2 * p
            dA = load(w0, 0)
            dB = load(w0 + 1, 2)
            dA.wait()
            stA = compute_store(w0, 0)
            dB.wait()
            stB = compute_store(w0 + 1, 2)
            for d in stA + stB:
                d.wait()
            return 0
        lax.fori_loop(0, _NWCH // 2, pairf, 0)

    def _writeback(last, zdst):
        def load(w, k):
            hb = row0 + w * _WCH
            return [
                pltpu.async_copy(acc.at[pl.ds(hb, _WCH)], rows.at[k],
                                 gsems.at[k]),
                pltpu.async_copy(
                    out_sum.at[pl.ds(hb, _WCH), pl.ds(col0, _HALF)],
                    rows.at[k + 1], gsems.at[k + 1]),
            ]
        def compute_store(w, k):
            hb = row0 + w * _WCH
            a = rows.at[k]
            b = rows.at[k + 1]
            def gf(g, _):
                dvec = dinv[pl.ds(w * _WCH + g * 16, 16)]
                for r16 in range(16):
                    r = g * 16 + r16
                    di = dvec[r16]
                    s0 = b[r, 0:16] + a[r, 0:16] * di
                    s1 = b[r, 16:32] + a[r, 16:32] * di
                    if last:
                        b[r, 0:16] = s0 * 0.25
                        b[r, 16:32] = s1 * 0.25
                    else:
                        b[r, 0:16] = s0
                        b[r, 16:32] = s1
                        d2 = di * di
                        a[r, 0:16] = a[r, 0:16] * d2
                        a[r, 16:32] = a[r, 16:32] * d2
                return 0
            lax.fori_loop(0, _WCH // 16, gf, 0)
            st = [pltpu.async_copy(
                b, out_sum.at[pl.ds(hb, _WCH), pl.ds(col0, _HALF)],
                ssems.at[k])]
            if not last:
                st.append(pltpu.async_copy(a, zdst.at[pl.ds(zoff + hb, _WCH)],
                                           ssems.at[k + 1]))
            return st
        def pairf(p, _):
            w0 = 2 * p
            dA = load(w0, 0)
            dB = load(w0 + 1, 2)
            for d in dA:
                d.wait()
            stA = compute_store(w0, 0)
            for d in dB:
                d.wait()
            stB = compute_store(w0 + 1, 2)
            for d in stA + stB:
                d.wait()
            return 0
        lax.fori_loop(0, _NWCH // 2, pairf, 0)

    for k in range(_CHUNK // 16):
        ones1[pl.ds(k * 16, 16)] = f1
    def zf(g, _):
        dinv[pl.ds(g * 16, 16)] = f0
        return 0
    lax.fori_loop(0, _RPT // 16, zf, 0)
    pltpu.sync_copy(dinv, dacc.at[pl.ds(row0, _RPT)])
    plsc.subcore_barrier()
    _edge_pass(None)
    plsc.subcore_barrier()
    _dinv_phase()
    _z0_phase()

    zsrc = za
    for l in range(3):
        _clear_acc_slice()
        plsc.subcore_barrier()
        _edge_pass(zsrc)
        plsc.subcore_barrier()
        zdst = zb if zsrc is za else za
        _writeback(last=(l == 2), zdst=zdst)
        zsrc = zdst


_lgcn = functools.partial(
    pl.kernel,
    out_type=(
        jax.ShapeDtypeStruct((_NPAD, _D), jnp.float32),
        jax.ShapeDtypeStruct((2 * _NPAD, _HALF), jnp.float32),
        jax.ShapeDtypeStruct((2 * _NPAD, _HALF), jnp.float32),
    ),
    mesh=plsc.VectorSubcoreMesh(core_axis_name="c", subcore_axis_name="s"),
    compiler_params=pltpu.CompilerParams(use_tc_tiling_on_sc=False),
    scratch_types=[
        pltpu.VMEM_SHARED((_NPAD, _HALF), jnp.float32),
        pltpu.VMEM_SHARED((_NPAD,), jnp.float32),
        pltpu.VMEM((_CHUNK,), jnp.float32),
        pltpu.VMEM((2, _BLK, _CHUNK), jnp.int32),
        pltpu.VMEM((2, _BLK, _CHUNK), jnp.int32),
        pltpu.VMEM((_NBUF, _CHUNK, _HALF), jnp.float32),
        pltpu.VMEM((_RPT,), jnp.float32),
        pltpu.SemaphoreType.DMA((_NBUF,)),
        pltpu.SemaphoreType.DMA((_NBUF,)),
        pltpu.SemaphoreType.DMA((2,)),
    ],
)(_lgcn_body)


def kernel(user_table, item_table, edge_index):
    all_emb = jnp.concatenate([user_table, item_table], axis=0)
    x0 = jnp.pad(all_emb, ((0, _NPAD - _N), (0, 0)))
    nblk_tot = _NCHROWS // _BLK
    src3 = edge_index[0].reshape(nblk_tot, _BLK, _CHUNK)
    dst2 = edge_index[1].reshape(nblk_tot, _BLK, _CHUNK)
    out_sum, _, _ = _lgcn(x0, src3, dst2)
    final = out_sum[:_N]
    return final[:_NUM_USERS], final[_NUM_USERS:]


# speedup vs baseline: 1.3038x; 1.0728x over previous
import functools

import jax
import jax.numpy as jnp
from jax import lax
from jax.experimental import pallas as pl
from jax.experimental.pallas import tpu as pltpu
from jax.experimental.pallas import tpu_sc as plsc

_NUM_USERS = 25000
_NUM_ITEMS = 25000
_D = 64
_HALF = 32
_N = _NUM_USERS + _NUM_ITEMS
_E = 800000
_NS = 16
_NPAD = 51200
_RPT = _NPAD // _NS
_WCH = 80
_NWCH = _RPT // _WCH
_NBUF = 5
_CHUNK = 80
_EPT = _E // _NS
_BLK = 25
_NBLK = _EPT // (_CHUNK * _BLK)
_NCHROWS = _E // _CHUNK


def _lgcn_body(x0, src3, dst2, out_sum, za, zb,
               acc, dacc, ones1, srcb, dstb, rows, dinv,
               gsems, ssems, isems):
    c = lax.axis_index("c")
    s = lax.axis_index("s")
    row0 = s * _RPT
    zoff = c * _NPAD
    col0 = c * _HALF
    blk0 = s * _NBLK

    f1 = jnp.full((16,), 1.0, jnp.float32)
    f0 = jnp.zeros((16,), jnp.float32)

    def _clear_acc_slice():
        zbuf = rows.at[4]
        def zf(r, _):
            zbuf[r, 0:16] = f0
            zbuf[r, 16:32] = f0
            return 0
        lax.fori_loop(0, _WCH, zf, 0)
        def f(w, _):
            pltpu.sync_copy(zbuf, acc.at[pl.ds(row0 + w * _WCH, _WCH)])
            return 0
        lax.fori_loop(0, _NWCH, f, 0)

    def _edge_pass(zsrc):
        deg = zsrc is None
        ztab = None if deg else zsrc.at[pl.ds(zoff, _NPAD)]

        def load_idx(setk, b):
            ds_ = [pltpu.async_copy(dst2.at[blk0 + b], dstb.at[setk],
                                    isems.at[0])]
            if not deg:
                ds_.append(pltpu.async_copy(src3.at[blk0 + b], srcb.at[setk],
                                            isems.at[1]))
            return ds_

        def process(setk, b, last, idx_wait=()):
            sb = srcb.at[setk]
            db = dstb.at[setk]
            nsb = srcb.at[1 - setk]
            if deg:
                for d in idx_wait:
                    d.wait()
                descs = [pltpu.async_copy(ones1, dacc.at[db.at[j]],
                                          ssems.at[j % _NBUF], add=True)
                         for j in range(_BLK)]
                for d in descs:
                    d.wait()
                return
            def gather(j):
                return pltpu.async_copy(ztab.at[sb.at[j]],
                                        rows.at[j % _NBUF],
                                        gsems.at[j % _NBUF])
            def scatter(j):
                return pltpu.async_copy(rows.at[j % _NBUF],
                                        acc.at[db.at[j]],
                                        ssems.at[j % _NBUF], add=True)
            gd = {}
            sd = {}
            for j in range(_BLK):
                if j == _BLK - _NBUF:
                    for d in idx_wait:
                        d.wait()
                if j + _NBUF - 1 < _BLK:
                    if j >= 1:
                        sd[j - 1].wait()
                    gd[j + _NBUF - 1] = gather(j + _NBUF - 1)
                elif not last:
                    sd[j - 1].wait()
                    jn = j - (_BLK - _NBUF + 1)
                    pltpu.async_copy(ztab.at[nsb.at[jn]],
                                     rows.at[jn % _NBUF], gsems.at[jn % _NBUF])
                if j < _NBUF - 1:
                    pltpu.make_async_copy(ztab.at[sb.at[j]],
                                          rows.at[j % _NBUF],
                                          gsems.at[j % _NBUF]).wait()
                else:
                    gd[j].wait()
                sd[j] = scatter(j)
            if last:
                for j in range(_BLK - _NBUF, _BLK):
                    sd[j].wait()
            else:
                sd[_BLK - 1].wait()

        for d in load_idx(0, 0):
            d.wait()
        if not deg:
            sb0 = srcb.at[0]
            for j in range(_NBUF - 1):
                pltpu.async_copy(ztab.at[sb0.at[j]], rows.at[j], gsems.at[j])
        def pair(p, _):
            b0 = 2 * p
            d1 = load_idx(1, b0 + 1)
            process(0, b0, last=False, idx_wait=d1)
            d0 = load_idx(0, b0 + 2)
            process(1, b0 + 1, last=False, idx_wait=d0)
            return 0
        lax.fori_loop(0, _NBLK // 2, pair, 0)
        process(0, _NBLK - 1, last=True)

    def _dinv_phase():
        magic = jnp.full((16,), 0x5F3759DF, jnp.int32)
        one_i = jnp.full((16,), 1, jnp.int32)
        pltpu.sync_copy(dacc.at[pl.ds(row0, _RPT)], dinv)
        def gf(g, _):
            d = dinv[pl.ds(g * 16, 16)]
            ib = lax.bitcast_convert_type(d, jnp.int32)
            y = lax.bitcast_convert_type(
                magic - lax.shift_right_logical(ib, one_i), jnp.float32)
            for _i in range(4):
                y = y * (1.5 - 0.5 * d * y * y)
            y = jnp.where(d > 0.5, y, 0.0)
            dinv[pl.ds(g * 16, 16)] = y
            return 0
        lax.fori_loop(0, _RPT // 16, gf, 0)

    def _z0_phase():
        def load(w, k):
            hb = row0 + w * _WCH
            return pltpu.async_copy(
                x0.at[pl.ds(hb, _WCH), pl.ds(col0, _HALF)], rows.at[k],
                gsems.at[k])
        def compute_store(w, k):
            hb = row0 + w * _WCH
            a = rows.at[k]
            z = rows.at[k + 1]
            def gf(g, _):
                dvec = dinv[pl.ds(w * _WCH + g * 16, 16)]
                for r16 in range(16):
                    r = g * 16 + r16
                    di = dvec[r16]
                    z[r, 0:16] = a[r, 0:16] * di
                    z[r, 16:32] = a[r, 16:32] * di
                return 0
            lax.fori_loop(0, _WCH // 16, gf, 0)
            return [
                pltpu.async_copy(
                    a, out_sum.at[pl.ds(hb, _WCH), pl.ds(col0, _HALF)],
                    ssems.at[k]),
                pltpu.async_copy(z, za.at[pl.ds(zoff + hb, _WCH)],
                                 ssems.at[k + 1]),
            ]
        def pairf(p, _):
            w0 = 2 * p
            dA = load(w0, 0)
            dB = load(w0 + 1, 2)
            dA.wait()
            stA = compute_store(w0, 0)
            dB.wait()
            stB = compute_store(w0 + 1, 2)
            for d in stA + stB:
                d.wait()
            return 0
        lax.fori_loop(0, _NWCH // 2, pairf, 0)

    def _writeback(last, zdst):
        def load(w, k):
            hb = row0 + w * _WCH
            return [
                pltpu.async_copy(acc.at[pl.ds(hb, _WCH)], rows.at[k],
                                 gsems.at[k]),
                pltpu.async_copy(
                    out_sum.at[pl.ds(hb, _WCH), pl.ds(col0, _HALF)],
                    rows.at[k + 1], gsems.at[k + 1]),
            ]
        def compute_store(w, k):
            hb = row0 + w * _WCH
            a = rows.at[k]
            b = rows.at[k + 1]
            def gf(g, _):
                dvec = dinv[pl.ds(w * _WCH + g * 16, 16)]
                for r16 in range(16):
                    r = g * 16 + r16
                    di = dvec[r16]
                    s0 = b[r, 0:16] + a[r, 0:16] * di
                    s1 = b[r, 16:32] + a[r, 16:32] * di
                    if last:
                        b[r, 0:16] = s0 * 0.25
                        b[r, 16:32] = s1 * 0.25
                    else:
                        b[r, 0:16] = s0
                        b[r, 16:32] = s1
                        d2 = di * di
                        a[r, 0:16] = a[r, 0:16] * d2
                        a[r, 16:32] = a[r, 16:32] * d2
                return 0
            lax.fori_loop(0, _WCH // 16, gf, 0)
            st = [pltpu.async_copy(
                b, out_sum.at[pl.ds(hb, _WCH), pl.ds(col0, _HALF)],
                ssems.at[k])]
            if not last:
                st.append(pltpu.async_copy(a, zdst.at[pl.ds(zoff + hb, _WCH)],
                                           ssems.at[k + 1]))
            return st
        def pairf(p, _):
            w0 = 2 * p
            dA = load(w0, 0)
            dB = load(w0 + 1, 2)
            for d in dA:
                d.wait()
            stA = compute_store(w0, 0)
            for d in dB:
                d.wait()
            stB = compute_store(w0 + 1, 2)
            for d in stA + stB:
                d.wait()
            return 0
        lax.fori_loop(0, _NWCH // 2, pairf, 0)

    for k in range(_CHUNK // 16):
        ones1[pl.ds(k * 16, 16)] = f1
    def zf(g, _):
        dinv[pl.ds(g * 16, 16)] = f0
        return 0
    lax.fori_loop(0, _RPT // 16, zf, 0)
    pltpu.sync_copy(dinv, dacc.at[pl.ds(row0, _RPT)])
    plsc.subcore_barrier()
    _edge_pass(None)
    plsc.subcore_barrier()
    _dinv_phase()
    _z0_phase()

    zsrc = za
    for l in range(3):
        _clear_acc_slice()
        plsc.subcore_barrier()
        _edge_pass(zsrc)
        plsc.subcore_barrier()
        zdst = zb if zsrc is za else za
        _writeback(last=(l == 2), zdst=zdst)
        zsrc = zdst


_lgcn = functools.partial(
    pl.kernel,
    out_type=(
        jax.ShapeDtypeStruct((_NPAD, _D), jnp.float32),
        jax.ShapeDtypeStruct((2 * _NPAD, _HALF), jnp.float32),
        jax.ShapeDtypeStruct((2 * _NPAD, _HALF), jnp.float32),
    ),
    mesh=plsc.VectorSubcoreMesh(core_axis_name="c", subcore_axis_name="s"),
    compiler_params=pltpu.CompilerParams(use_tc_tiling_on_sc=False),
    scratch_types=[
        pltpu.VMEM_SHARED((_NPAD, _HALF), jnp.float32),
        pltpu.VMEM_SHARED((_NPAD,), jnp.float32),
        pltpu.VMEM((_CHUNK,), jnp.float32),
        pltpu.VMEM((2, _BLK, _CHUNK), jnp.int32),
        pltpu.VMEM((2, _BLK, _CHUNK), jnp.int32),
        pltpu.VMEM((_NBUF, _CHUNK, _HALF), jnp.float32),
        pltpu.VMEM((_RPT,), jnp.float32),
        pltpu.SemaphoreType.DMA((_NBUF,)),
        pltpu.SemaphoreType.DMA((_NBUF,)),
        pltpu.SemaphoreType.DMA((2,)),
    ],
)(_lgcn_body)


def kernel(user_table, item_table, edge_index):
    all_emb = jnp.concatenate([user_table, item_table], axis=0)
    x0 = jnp.pad(all_emb, ((0, _NPAD - _N), (0, 0)))
    nblk_tot = _NCHROWS // _BLK
    src3 = edge_index[0].reshape(nblk_tot, _BLK, _CHUNK)
    dst2 = edge_index[1].reshape(nblk_tot, _BLK, _CHUNK)
    out_sum, _, _ = _lgcn(x0, src3, dst2)
    final = out_sum[:_N]
    return final[:_NUM_USERS], final[_NUM_USERS:]


# speedup vs baseline: 1.3170x; 1.0101x over previous
import functools

import jax
import jax.numpy as jnp
from jax import lax
from jax.experimental import pallas as pl
from jax.experimental.pallas import tpu as pltpu
from jax.experimental.pallas import tpu_sc as plsc

_NUM_USERS = 25000
_NUM_ITEMS = 25000
_D = 64
_HALF = 32
_N = _NUM_USERS + _NUM_ITEMS
_E = 800000
_NS = 16
_NPAD = 51200
_RPT = _NPAD // _NS
_WCH = 80
_NWCH = _RPT // _WCH
_NBUF = 5
_CHUNK = 80
_EPT = _E // _NS
_BLK = 25
_NBLK = _EPT // (_CHUNK * _BLK)
_NCHROWS = _E // _CHUNK


def _lgcn_body(x0, src3, dst2, out_sum, za, zb,
               acc, dacc, ones1, srcb, dstb, rows, dinv,
               gsems, ssems, isems):
    c = lax.axis_index("c")
    s = lax.axis_index("s")
    row0 = s * _RPT
    zoff = c * _NPAD
    col0 = c * _HALF
    blk0 = s * _NBLK

    f1 = jnp.full((16,), 1.0, jnp.float32)
    f0 = jnp.zeros((16,), jnp.float32)

    def _clear_acc_slice():
        zbuf = rows.at[4]
        def zf(r, _):
            zbuf[r, 0:16] = f0
            zbuf[r, 16:32] = f0
            return 0
        lax.fori_loop(0, _WCH, zf, 0)
        def f(w, _):
            pltpu.sync_copy(zbuf, acc.at[pl.ds(row0 + w * _WCH, _WCH)])
            return 0
        lax.fori_loop(0, _NWCH, f, 0)

    def _edge_pass(zsrc):
        deg = zsrc is None
        ztab = None if deg else zsrc.at[pl.ds(zoff, _NPAD)]

        def load_idx(setk, b):
            ds_ = [pltpu.async_copy(dst2.at[blk0 + b], dstb.at[setk],
                                    isems.at[0])]
            if not deg:
                ds_.append(pltpu.async_copy(src3.at[blk0 + b], srcb.at[setk],
                                            isems.at[1]))
            return ds_

        def process(setk, b, last, idx_wait=()):
            sb = srcb.at[setk]
            db = dstb.at[setk]
            nsb = srcb.at[1 - setk]
            if deg:
                for d in idx_wait:
                    d.wait()
                descs = [pltpu.async_copy(ones1, dacc.at[db.at[j]],
                                          ssems.at[j % _NBUF], add=True)
                         for j in range(_BLK)]
                for d in descs:
                    d.wait()
                return
            def gather(j):
                return pltpu.async_copy(ztab.at[sb.at[j]],
                                        rows.at[j % _NBUF],
                                        gsems.at[j % _NBUF])
            def scatter(j):
                return pltpu.async_copy(rows.at[j % _NBUF],
                                        acc.at[db.at[j]],
                                        ssems.at[j % _NBUF], add=True)
            gd = {}
            sd = {}
            for j in range(_BLK):
                if j == _BLK - _NBUF:
                    for d in idx_wait:
                        d.wait()
                if j + _NBUF - 1 < _BLK:
                    if j >= 1:
                        sd[j - 1].wait()
                    gd[j + _NBUF - 1] = gather(j + _NBUF - 1)
                elif not last:
                    sd[j - 1].wait()
                    jn = j - (_BLK - _NBUF + 1)
                    pltpu.async_copy(ztab.at[nsb.at[jn]],
                                     rows.at[jn % _NBUF], gsems.at[jn % _NBUF])
                if j < _NBUF - 1:
                    pltpu.make_async_copy(ztab.at[sb.at[j]],
                                          rows.at[j % _NBUF],
                                          gsems.at[j % _NBUF]).wait()
                else:
                    gd[j].wait()
                sd[j] = scatter(j)
            if last:
                for j in range(_BLK - _NBUF, _BLK):
                    sd[j].wait()
            else:
                sd[_BLK - 1].wait()

        for d in load_idx(0, 0):
            d.wait()
        if not deg:
            sb0 = srcb.at[0]
            for j in range(_NBUF - 1):
                pltpu.async_copy(ztab.at[sb0.at[j]], rows.at[j], gsems.at[j])
        def pair(p, _):
            b0 = 2 * p
            d1 = load_idx(1, b0 + 1)
            process(0, b0, last=False, idx_wait=d1)
            d0 = load_idx(0, b0 + 2)
            process(1, b0 + 1, last=False, idx_wait=d0)
            return 0
        lax.fori_loop(0, _NBLK // 2, pair, 0)
        process(0, _NBLK - 1, last=True)

    def _dinv_phase():
        magic = jnp.full((16,), 0x5F3759DF, jnp.int32)
        one_i = jnp.full((16,), 1, jnp.int32)
        pltpu.sync_copy(dacc.at[pl.ds(row0, _RPT)], dinv)
        def gf(g, _):
            d = dinv[pl.ds(g * 16, 16)]
            ib = lax.bitcast_convert_type(d, jnp.int32)
            y = lax.bitcast_convert_type(
                magic - lax.shift_right_logical(ib, one_i), jnp.float32)
            for _i in range(4):
                y = y * (1.5 - 0.5 * d * y * y)
            y = jnp.where(d > 0.5, y, 0.0)
            dinv[pl.ds(g * 16, 16)] = y
            return 0
        lax.fori_loop(0, _RPT // 16, gf, 0)

    def _z0_phase():
        def load(w, k):
            hb = row0 + w * _WCH
            return pltpu.async_copy(
                x0.at[pl.ds(hb, _WCH), pl.ds(col0, _HALF)], rows.at[k],
                gsems.at[k])
        def compute_store(w, k):
            hb = row0 + w * _WCH
            a = rows.at[k]
            z = rows.at[k + 1]
            def gf(g, _):
                dvec = dinv[pl.ds(w * _WCH + g * 16, 16)]
                for r16 in range(16):
                    r = g * 16 + r16
                    di = dvec[r16]
                    z[r, 0:16] = a[r, 0:16] * di
                    z[r, 16:32] = a[r, 16:32] * di
                return 0
            lax.fori_loop(0, _WCH // 16, gf, 0)
            return [
                pltpu.async_copy(
                    a, out_sum.at[pl.ds(hb, _WCH), pl.ds(col0, _HALF)],
                    ssems.at[k]),
                pltpu.async_copy(z, za.at[pl.ds(zoff + hb, _WCH)],
                                 ssems.at[k + 1]),
            ]
        def rw(i):
            pltpu.make_async_copy(rows.at[i], za.at[pl.ds(zoff + row0, _WCH)],
                                  ssems.at[i]).wait()
        for i in range(4):
            pltpu.async_copy(rows.at[i], zb.at[pl.ds(zoff + row0, _WCH)],
                             ssems.at[i])
        def pf(p, _):
            w0 = 2 * p
            rw(0)
            dA = load(w0, 0)
            rw(2)
            dB = load(w0 + 1, 2)
            dA.wait()
            rw(1)
            compute_store(w0, 0)
            dB.wait()
            rw(3)
            compute_store(w0 + 1, 2)
            return 0
        lax.fori_loop(0, _NWCH // 2, pf, 0)
        for i in range(4):
            rw(i)

    def _writeback(last, zdst):
        def load(w, k):
            hb = row0 + w * _WCH
            return [
                pltpu.async_copy(acc.at[pl.ds(hb, _WCH)], rows.at[k],
                                 gsems.at[k]),
                pltpu.async_copy(
                    out_sum.at[pl.ds(hb, _WCH), pl.ds(col0, _HALF)],
                    rows.at[k + 1], gsems.at[k + 1]),
            ]
        def compute_store(w, k):
            hb = row0 + w * _WCH
            a = rows.at[k]
            b = rows.at[k + 1]
            def gf(g, _):
                dvec = dinv[pl.ds(w * _WCH + g * 16, 16)]
                for r16 in range(16):
                    r = g * 16 + r16
                    di = dvec[r16]
                    s0 = b[r, 0:16] + a[r, 0:16] * di
                    s1 = b[r, 16:32] + a[r, 16:32] * di
                    if last:
                        b[r, 0:16] = s0 * 0.25
                        b[r, 16:32] = s1 * 0.25
                    else:
                        b[r, 0:16] = s0
                        b[r, 16:32] = s1
                        d2 = di * di
                        a[r, 0:16] = a[r, 0:16] * d2
                        a[r, 16:32] = a[r, 16:32] * d2
                return 0
            lax.fori_loop(0, _WCH // 16, gf, 0)
            st = [pltpu.async_copy(
                b, out_sum.at[pl.ds(hb, _WCH), pl.ds(col0, _HALF)],
                ssems.at[k])]
            if not last:
                st.append(pltpu.async_copy(a, zdst.at[pl.ds(zoff + hb, _WCH)],
                                           ssems.at[k + 1]))
            return st
        def rw(i):
            pltpu.make_async_copy(rows.at[i], za.at[pl.ds(zoff + row0, _WCH)],
                                  ssems.at[i]).wait()
        def entry_waits(k):
            rw(k)
            if not last:
                rw(k + 1)
        dead = za if zdst is zb else zb
        primed = (0, 2) if last else (0, 1, 2, 3)
        for i in primed:
            pltpu.async_copy(rows.at[i], dead.at[pl.ds(zoff + row0, _WCH)],
                             ssems.at[i])
        def pf(p, _):
            w0 = 2 * p
            entry_waits(0)
            dA = load(w0, 0)
            entry_waits(2)
            dB = load(w0 + 1, 2)
            for d in dA:
                d.wait()
            compute_store(w0, 0)
            for d in dB:
                d.wait()
            compute_store(w0 + 1, 2)
            return 0
        lax.fori_loop(0, _NWCH // 2, pf, 0)
        for k in (0, 2):
            entry_waits(k)

    for k in range(_CHUNK // 16):
        ones1[pl.ds(k * 16, 16)] = f1
    def zf(g, _):
        dinv[pl.ds(g * 16, 16)] = f0
        return 0
    lax.fori_loop(0, _RPT // 16, zf, 0)
    pltpu.sync_copy(dinv, dacc.at[pl.ds(row0, _RPT)])
    plsc.subcore_barrier()
    _edge_pass(None)
    plsc.subcore_barrier()
    _dinv_phase()
    _z0_phase()

    zsrc = za
    for l in range(3):
        _clear_acc_slice()
        plsc.subcore_barrier()
        _edge_pass(zsrc)
        plsc.subcore_barrier()
        zdst = zb if zsrc is za else za
        _writeback(last=(l == 2), zdst=zdst)
        zsrc = zdst


_lgcn = functools.partial(
    pl.kernel,
    out_type=(
        jax.ShapeDtypeStruct((_NPAD, _D), jnp.float32),
        jax.ShapeDtypeStruct((2 * _NPAD, _HALF), jnp.float32),
        jax.ShapeDtypeStruct((2 * _NPAD, _HALF), jnp.float32),
    ),
    mesh=plsc.VectorSubcoreMesh(core_axis_name="c", subcore_axis_name="s"),
    compiler_params=pltpu.CompilerParams(use_tc_tiling_on_sc=False),
    scratch_types=[
        pltpu.VMEM_SHARED((_NPAD, _HALF), jnp.float32),
        pltpu.VMEM_SHARED((_NPAD,), jnp.float32),
        pltpu.VMEM((_CHUNK,), jnp.float32),
        pltpu.VMEM((2, _BLK, _CHUNK), jnp.int32),
        pltpu.VMEM((2, _BLK, _CHUNK), jnp.int32),
        pltpu.VMEM((_NBUF, _CHUNK, _HALF), jnp.float32),
        pltpu.VMEM((_RPT,), jnp.float32),
        pltpu.SemaphoreType.DMA((_NBUF,)),
        pltpu.SemaphoreType.DMA((_NBUF,)),
        pltpu.SemaphoreType.DMA((2,)),
    ],
)(_lgcn_body)


def kernel(user_table, item_table, edge_index):
    all_emb = jnp.concatenate([user_table, item_table], axis=0)
    x0 = jnp.pad(all_emb, ((0, _NPAD - _N), (0, 0)))
    nblk_tot = _NCHROWS // _BLK
    src3 = edge_index[0].reshape(nblk_tot, _BLK, _CHUNK)
    dst2 = edge_index[1].reshape(nblk_tot, _BLK, _CHUNK)
    out_sum, _, _ = _lgcn(x0, src3, dst2)
    final = out_sum[:_N]
    return final[:_NUM_USERS], final[_NUM_USERS:]


# speedup vs baseline: 1.4034x; 1.0656x over previous
import functools

import jax
import jax.numpy as jnp
from jax import lax
from jax.experimental import pallas as pl
from jax.experimental.pallas import tpu as pltpu
from jax.experimental.pallas import tpu_sc as plsc

_NUM_USERS = 25000
_NUM_ITEMS = 25000
_D = 64
_HALF = 32
_N = _NUM_USERS + _NUM_ITEMS
_E = 800000
_NS = 16
_NPAD = 51200
_RPT = _NPAD // _NS
_WCH = 80
_NWCH = _RPT // _WCH
_NBUF = 5
_CHUNK = 80
_EPT = _E // _NS
_BLK = 25
_NBLK = _EPT // (_CHUNK * _BLK)
_NCHROWS = _E // _CHUNK


def _lgcn_body(utab, itab, src3, dst2, ou, oi, opad, za, zb,
               acc, dacc, ones1, srcb, dstb, rows, dinv,
               gsems, ssems, isems):
    c = lax.axis_index("c")
    s = lax.axis_index("s")
    row0 = s * _RPT
    zoff = c * _NPAD
    col0 = c * _HALF
    blk0 = s * _NBLK

    f1 = jnp.full((16,), 1.0, jnp.float32)
    f0 = jnp.zeros((16,), jnp.float32)
    _B = _NUM_USERS

    def _sum_io(hb, buf, sem, write):
        def mk(tab, off, b0, n):
            t = tab.at[pl.ds(off, n), pl.ds(col0, _HALF)]
            bb = buf.at[pl.ds(b0, n)]
            if write:
                pltpu.async_copy(bb, t, sem)
            else:
                pltpu.async_copy(t, bb, sem)
        @pl.when(hb + _WCH <= _B)
        def _():
            mk(ou, hb, 0, _WCH)
        @pl.when((hb >= _B) & (hb + _WCH <= 2 * _B))
        def _():
            mk(oi, hb - _B, 0, _WCH)
        @pl.when(hb >= 2 * _B)
        def _():
            mk(opad, hb - 2 * _B, 0, _WCH)
        @pl.when((hb < _B) & (hb + _WCH > _B))
        def _():
            mk(ou, hb, 0, _WCH // 2)
            mk(oi, 0, _WCH // 2, _WCH // 2)

    def _x0_load(hb, buf, sem):
        def mk(tab, off, b0, n):
            pltpu.async_copy(tab.at[pl.ds(off, n), pl.ds(col0, _HALF)],
                             buf.at[pl.ds(b0, n)], sem)
        @pl.when(hb + _WCH <= _B)
        def _():
            mk(utab, hb, 0, _WCH)
        @pl.when((hb >= _B) & (hb + _WCH <= 2 * _B))
        def _():
            mk(itab, hb - _B, 0, _WCH)
        @pl.when(hb >= 2 * _B)
        def _():
            mk(utab, 0, 0, _WCH)
        @pl.when((hb < _B) & (hb + _WCH > _B))
        def _():
            mk(utab, hb, 0, _WCH // 2)
            mk(itab, 0, _WCH // 2, _WCH // 2)

    def _gw(k):
        pltpu.make_async_copy(utab.at[pl.ds(0, _WCH), pl.ds(col0, _HALF)],
                              rows.at[k], gsems.at[k]).wait()

    def _clear_acc_slice():
        zbuf = rows.at[4]
        def zf(r, _):
            zbuf[r, 0:16] = f0
            zbuf[r, 16:32] = f0
            return 0
        lax.fori_loop(0, _WCH, zf, 0)
        def f(w, _):
            pltpu.sync_copy(zbuf, acc.at[pl.ds(row0 + w * _WCH, _WCH)])
            return 0
        lax.fori_loop(0, _NWCH, f, 0)

    def _edge_pass(zsrc):
        deg = zsrc is None
        ztab = None if deg else zsrc.at[pl.ds(zoff, _NPAD)]

        def load_idx(setk, b):
            ds_ = [pltpu.async_copy(dst2.at[blk0 + b], dstb.at[setk],
                                    isems.at[0])]
            if not deg:
                ds_.append(pltpu.async_copy(src3.at[blk0 + b], srcb.at[setk],
                                            isems.at[1]))
            return ds_

        def process(setk, b, last, idx_wait=()):
            sb = srcb.at[setk]
            db = dstb.at[setk]
            nsb = srcb.at[1 - setk]
            if deg:
                for d in idx_wait:
                    d.wait()
                descs = [pltpu.async_copy(ones1, dacc.at[db.at[j]],
                                          ssems.at[j % _NBUF], add=True)
                         for j in range(_BLK)]
                for d in descs:
                    d.wait()
                return
            def gather(j):
                return pltpu.async_copy(ztab.at[sb.at[j]],
                                        rows.at[j % _NBUF],
                                        gsems.at[j % _NBUF])
            def scatter(j):
                return pltpu.async_copy(rows.at[j % _NBUF],
                                        acc.at[db.at[j]],
                                        ssems.at[j % _NBUF], add=True)
            gd = {}
            sd = {}
            for j in range(_BLK):
                if j == _BLK - _NBUF:
                    for d in idx_wait:
                        d.wait()
                if j + _NBUF - 1 < _BLK:
                    if j >= 1:
                        sd[j - 1].wait()
                    gd[j + _NBUF - 1] = gather(j + _NBUF - 1)
                elif not last:
                    sd[j - 1].wait()
                    jn = j - (_BLK - _NBUF + 1)
                    pltpu.async_copy(ztab.at[nsb.at[jn]],
                                     rows.at[jn % _NBUF], gsems.at[jn % _NBUF])
                if j < _NBUF - 1:
                    pltpu.make_async_copy(ztab.at[sb.at[j]],
                                          rows.at[j % _NBUF],
                                          gsems.at[j % _NBUF]).wait()
                else:
                    gd[j].wait()
                sd[j] = scatter(j)
            if last:
                for j in range(_BLK - _NBUF, _BLK):
                    sd[j].wait()
            else:
                sd[_BLK - 1].wait()

        for d in load_idx(0, 0):
            d.wait()
        if not deg:
            sb0 = srcb.at[0]
            for j in range(_NBUF - 1):
                pltpu.async_copy(ztab.at[sb0.at[j]], rows.at[j], gsems.at[j])
        def pair(p, _):
            b0 = 2 * p
            d1 = load_idx(1, b0 + 1)
            process(0, b0, last=False, idx_wait=d1)
            d0 = load_idx(0, b0 + 2)
            process(1, b0 + 1, last=False, idx_wait=d0)
            return 0
        lax.fori_loop(0, _NBLK // 2, pair, 0)
        process(0, _NBLK - 1, last=True)

    def _dinv_phase():
        magic = jnp.full((16,), 0x5F3759DF, jnp.int32)
        one_i = jnp.full((16,), 1, jnp.int32)
        pltpu.sync_copy(dacc.at[pl.ds(row0, _RPT)], dinv)
        def gf(g, _):
            d = dinv[pl.ds(g * 16, 16)]
            ib = lax.bitcast_convert_type(d, jnp.int32)
            y = lax.bitcast_convert_type(
                magic - lax.shift_right_logical(ib, one_i), jnp.float32)
            for _i in range(4):
                y = y * (1.5 - 0.5 * d * y * y)
            y = jnp.where(d > 0.5, y, 0.0)
            dinv[pl.ds(g * 16, 16)] = y
            return 0
        lax.fori_loop(0, _RPT // 16, gf, 0)

    def _z0_phase():
        def load(w, k):
            hb = row0 + w * _WCH
            _x0_load(hb, rows.at[k], gsems.at[k])
        def compute_store(w, k):
            hb = row0 + w * _WCH
            a = rows.at[k]
            z = rows.at[k + 1]
            def gf(g, _):
                dvec = dinv[pl.ds(w * _WCH + g * 16, 16)]
                for r16 in range(16):
                    r = g * 16 + r16
                    di = dvec[r16]
                    z[r, 0:16] = a[r, 0:16] * di
                    z[r, 16:32] = a[r, 16:32] * di
                return 0
            lax.fori_loop(0, _WCH // 16, gf, 0)
            _sum_io(hb, a, ssems.at[k], write=True)
            pltpu.async_copy(z, za.at[pl.ds(zoff + hb, _WCH)],
                             ssems.at[k + 1])
        def rw(i):
            pltpu.make_async_copy(rows.at[i], za.at[pl.ds(zoff + row0, _WCH)],
                                  ssems.at[i]).wait()
        for i in range(4):
            pltpu.async_copy(rows.at[i], zb.at[pl.ds(zoff + row0, _WCH)],
                             ssems.at[i])
        def pf(p, _):
            w0 = 2 * p
            rw(0)
            load(w0, 0)
            rw(2)
            load(w0 + 1, 2)
            _gw(0)
            rw(1)
            compute_store(w0, 0)
            _gw(2)
            rw(3)
            compute_store(w0 + 1, 2)
            return 0
        lax.fori_loop(0, _NWCH // 2, pf, 0)
        for i in range(4):
            rw(i)

    def _writeback(last, zdst):
        def load(w, k):
            hb = row0 + w * _WCH
            pltpu.async_copy(acc.at[pl.ds(hb, _WCH)], rows.at[k],
                             gsems.at[k])
            _sum_io(hb, rows.at[k + 1], gsems.at[k + 1], write=False)
        def compute_store(w, k):
            hb = row0 + w * _WCH
            a = rows.at[k]
            b = rows.at[k + 1]
            def gf(g, _):
                dvec = dinv[pl.ds(w * _WCH + g * 16, 16)]
                for r16 in range(16):
                    r = g * 16 + r16
                    di = dvec[r16]
                    s0 = b[r, 0:16] + a[r, 0:16] * di
                    s1 = b[r, 16:32] + a[r, 16:32] * di
                    if last:
                        b[r, 0:16] = s0 * 0.25
                        b[r, 16:32] = s1 * 0.25
                    else:
                        b[r, 0:16] = s0
                        b[r, 16:32] = s1
                        d2 = di * di
                        a[r, 0:16] = a[r, 0:16] * d2
                        a[r, 16:32] = a[r, 16:32] * d2
                return 0
            lax.fori_loop(0, _WCH // 16, gf, 0)
            _sum_io(hb, b, ssems.at[k], write=True)
            if not last:
                pltpu.async_copy(a, zdst.at[pl.ds(zoff + hb, _WCH)],
                                 ssems.at[k + 1])
        def rw(i):
            pltpu.make_async_copy(rows.at[i], za.at[pl.ds(zoff + row0, _WCH)],
                                  ssems.at[i]).wait()
        def entry_waits(k):
            rw(k)
            if not last:
                rw(k + 1)
        dead = za if zdst is zb else zb
        primed = (0, 2) if last else (0, 1, 2, 3)
        for i in primed:
            pltpu.async_copy(rows.at[i], dead.at[pl.ds(zoff + row0, _WCH)],
                             ssems.at[i])
        def pf(p, _):
            w0 = 2 * p
            entry_waits(0)
            load(w0, 0)
            entry_waits(2)
            load(w0 + 1, 2)
            _gw(0)
            _gw(1)
            compute_store(w0, 0)
            _gw(2)
            _gw(3)
            compute_store(w0 + 1, 2)
            return 0
        lax.fori_loop(0, _NWCH // 2, pf, 0)
        for k in (0, 2):
            entry_waits(k)

    for k in range(_CHUNK // 16):
        ones1[pl.ds(k * 16, 16)] = f1
    def zf(g, _):
        dinv[pl.ds(g * 16, 16)] = f0
        return 0
    lax.fori_loop(0, _RPT // 16, zf, 0)
    pltpu.sync_copy(dinv, dacc.at[pl.ds(row0, _RPT)])
    plsc.subcore_barrier()
    _edge_pass(None)
    plsc.subcore_barrier()
    _dinv_phase()
    _z0_phase()

    zsrc = za
    for l in range(3):
        _clear_acc_slice()
        plsc.subcore_barrier()
        _edge_pass(zsrc)
        plsc.subcore_barrier()
        zdst = zb if zsrc is za else za
        _writeback(last=(l == 2), zdst=zdst)
        zsrc = zdst


_lgcn = functools.partial(
    pl.kernel,
    out_type=(
        jax.ShapeDtypeStruct((_NUM_USERS, _D), jnp.float32),
        jax.ShapeDtypeStruct((_NUM_ITEMS, _D), jnp.float32),
        jax.ShapeDtypeStruct((_NPAD - _N, _D), jnp.float32),
        jax.ShapeDtypeStruct((2 * _NPAD, _HALF), jnp.float32),
        jax.ShapeDtypeStruct((2 * _NPAD, _HALF), jnp.float32),
    ),
    mesh=plsc.VectorSubcoreMesh(core_axis_name="c", subcore_axis_name="s"),
    compiler_params=pltpu.CompilerParams(use_tc_tiling_on_sc=False),
    scratch_types=[
        pltpu.VMEM_SHARED((_NPAD, _HALF), jnp.float32),
        pltpu.VMEM_SHARED((_NPAD,), jnp.float32),
        pltpu.VMEM((_CHUNK,), jnp.float32),
        pltpu.VMEM((2, _BLK, _CHUNK), jnp.int32),
        pltpu.VMEM((2, _BLK, _CHUNK), jnp.int32),
        pltpu.VMEM((_NBUF, _CHUNK, _HALF), jnp.float32),
        pltpu.VMEM((_RPT,), jnp.float32),
        pltpu.SemaphoreType.DMA((_NBUF,)),
        pltpu.SemaphoreType.DMA((_NBUF,)),
        pltpu.SemaphoreType.DMA((2,)),
    ],
)(_lgcn_body)


def kernel(user_table, item_table, edge_index):
    nblk_tot = _NCHROWS // _BLK
    src3 = edge_index[0].reshape(nblk_tot, _BLK, _CHUNK)
    dst2 = edge_index[1].reshape(nblk_tot, _BLK, _CHUNK)
    user_final, item_final, _, _, _ = _lgcn(user_table, item_table, src3, dst2)
    return user_final, item_final


# speedup vs baseline: 1.4037x; 1.0002x over previous
import functools

import jax
import jax.numpy as jnp
from jax import lax
from jax.experimental import pallas as pl
from jax.experimental.pallas import tpu as pltpu
from jax.experimental.pallas import tpu_sc as plsc

_NUM_USERS = 25000
_NUM_ITEMS = 25000
_D = 64
_HALF = 32
_N = _NUM_USERS + _NUM_ITEMS
_E = 800000
_NS = 16
_NPAD = 51200
_RPT = _NPAD // _NS
_WCH = 80
_NWCH = _RPT // _WCH
_NBUF = 5
_CHUNK = 80
_EPT = _E // _NS
_BLK = 25
_NBLK = _EPT // (_CHUNK * _BLK)
_NCHROWS = _E // _CHUNK


def _lgcn_body(utab, itab, src3, dst2, ou, oi, opad, za, zb,
               acc, dacc, ones1, srcb, dstb, rows, dinv,
               gsems, ssems, isems):
    c = lax.axis_index("c")
    s = lax.axis_index("s")
    row0 = s * _RPT
    zoff = c * _NPAD
    col0 = c * _HALF
    blk0 = s * _NBLK

    f1 = jnp.full((16,), 1.0, jnp.float32)
    f0 = jnp.zeros((16,), jnp.float32)
    _B = _NUM_USERS

    def _sum_io(hb, buf, sem, write):
        def mk(tab, off, b0, n):
            t = tab.at[pl.ds(off, n), pl.ds(col0, _HALF)]
            bb = buf.at[pl.ds(b0, n)]
            if write:
                pltpu.async_copy(bb, t, sem)
            else:
                pltpu.async_copy(t, bb, sem)
        @pl.when(hb + _WCH <= _B)
        def _():
            mk(ou, hb, 0, _WCH)
        @pl.when((hb >= _B) & (hb + _WCH <= 2 * _B))
        def _():
            mk(oi, hb - _B, 0, _WCH)
        @pl.when(hb >= 2 * _B)
        def _():
            mk(opad, hb - 2 * _B, 0, _WCH)
        @pl.when((hb < _B) & (hb + _WCH > _B))
        def _():
            mk(ou, hb, 0, _WCH // 2)
            mk(oi, 0, _WCH // 2, _WCH // 2)

    def _x0_load(hb, buf, sem):
        def mk(tab, off, b0, n):
            pltpu.async_copy(tab.at[pl.ds(off, n), pl.ds(col0, _HALF)],
                             buf.at[pl.ds(b0, n)], sem)
        @pl.when(hb + _WCH <= _B)
        def _():
            mk(utab, hb, 0, _WCH)
        @pl.when((hb >= _B) & (hb + _WCH <= 2 * _B))
        def _():
            mk(itab, hb - _B, 0, _WCH)
        @pl.when(hb >= 2 * _B)
        def _():
            mk(utab, 0, 0, _WCH)
        @pl.when((hb < _B) & (hb + _WCH > _B))
        def _():
            mk(utab, hb, 0, _WCH // 2)
            mk(itab, 0, _WCH // 2, _WCH // 2)

    def _gw(k):
        pltpu.make_async_copy(utab.at[pl.ds(0, _WCH), pl.ds(col0, _HALF)],
                              rows.at[k], gsems.at[k]).wait()

    def _clear_acc_slice():
        zbuf = rows.at[4]
        def zf(r, _):
            zbuf[r, 0:16] = f0
            zbuf[r, 16:32] = f0
            return 0
        lax.fori_loop(0, _WCH, zf, 0)
        def f(w, _):
            pltpu.sync_copy(zbuf, acc.at[pl.ds(row0 + w * _WCH, _WCH)])
            return 0
        lax.fori_loop(0, _NWCH, f, 0)

    def _edge_pass(zsrc):
        deg = zsrc is None
        ztab = None if deg else zsrc.at[pl.ds(zoff, _NPAD)]

        def load_idx(setk, b):
            ds_ = [pltpu.async_copy(dst2.at[blk0 + b], dstb.at[setk],
                                    isems.at[0])]
            if not deg:
                ds_.append(pltpu.async_copy(src3.at[blk0 + b], srcb.at[setk],
                                            isems.at[1]))
            return ds_

        def process(setk, b, last, idx_wait=()):
            sb = srcb.at[setk]
            db = dstb.at[setk]
            nsb = srcb.at[1 - setk]
            if deg:
                for d in idx_wait:
                    d.wait()
                descs = [pltpu.async_copy(ones1, dacc.at[db.at[j]],
                                          ssems.at[j % _NBUF], add=True)
                         for j in range(_BLK)]
                for d in descs:
                    d.wait()
                return
            def gather(j):
                return pltpu.async_copy(ztab.at[sb.at[j]],
                                        rows.at[j % _NBUF],
                                        gsems.at[j % _NBUF])
            def scatter(j):
                return pltpu.async_copy(rows.at[j % _NBUF],
                                        acc.at[db.at[j]],
                                        ssems.at[j % _NBUF], add=True)
            gd = {}
            sd = {}
            for j in range(_BLK):
                if j == _BLK - _NBUF:
                    for d in idx_wait:
                        d.wait()
                if j + _NBUF - 1 < _BLK:
                    if j >= 1:
                        sd[j - 1].wait()
                    gd[j + _NBUF - 1] = gather(j + _NBUF - 1)
                elif not last:
                    sd[j - 1].wait()
                    jn = j - (_BLK - _NBUF + 1)
                    pltpu.async_copy(ztab.at[nsb.at[jn]],
                                     rows.at[jn % _NBUF], gsems.at[jn % _NBUF])
                if j < _NBUF - 1:
                    pltpu.make_async_copy(ztab.at[sb.at[j]],
                                          rows.at[j % _NBUF],
                                          gsems.at[j % _NBUF]).wait()
                else:
                    gd[j].wait()
                sd[j] = scatter(j)
            if last:
                for j in range(_BLK - _NBUF, _BLK):
                    sd[j].wait()
            else:
                sd[_BLK - 1].wait()

        for d in load_idx(0, 0):
            d.wait()
        if not deg:
            sb0 = srcb.at[0]
            for j in range(_NBUF - 1):
                pltpu.async_copy(ztab.at[sb0.at[j]], rows.at[j], gsems.at[j])
        def pair(p, _):
            b0 = 2 * p
            d1 = load_idx(1, b0 + 1)
            process(0, b0, last=False, idx_wait=d1)
            d0 = load_idx(0, b0 + 2)
            process(1, b0 + 1, last=False, idx_wait=d0)
            return 0
        lax.fori_loop(0, _NBLK // 2, pair, 0)
        process(0, _NBLK - 1, last=True)

    def _dinv_phase():
        magic = jnp.full((16,), 0x5F3759DF, jnp.int32)
        one_i = jnp.full((16,), 1, jnp.int32)
        pltpu.sync_copy(dacc.at[pl.ds(row0, _RPT)], dinv)
        def gf(g, _):
            d = dinv[pl.ds(g * 16, 16)]
            ib = lax.bitcast_convert_type(d, jnp.int32)
            y = lax.bitcast_convert_type(
                magic - lax.shift_right_logical(ib, one_i), jnp.float32)
            for _i in range(4):
                y = y * (1.5 - 0.5 * d * y * y)
            y = jnp.where(d > 0.5, y, 0.0)
            dinv[pl.ds(g * 16, 16)] = y
            return 0
        lax.fori_loop(0, _RPT // 16, gf, 0)

    def _z0_phase():
        def load(w, k):
            hb = row0 + w * _WCH
            _x0_load(hb, rows.at[k], gsems.at[k])
        def compute_store(w, k):
            hb = row0 + w * _WCH
            a = rows.at[k]
            z = rows.at[k + 1]
            def gf(g, _):
                dvec = dinv[pl.ds(w * _WCH + g * 16, 16)]
                for r16 in range(16):
                    r = g * 16 + r16
                    di = dvec[r16]
                    z[r, 0:16] = a[r, 0:16] * di
                    z[r, 16:32] = a[r, 16:32] * di
                return 0
            lax.fori_loop(0, _WCH // 16, gf, 0)
            _sum_io(hb, a, ssems.at[k], write=True)
            pltpu.async_copy(z, za.at[pl.ds(zoff + hb, _WCH)],
                             ssems.at[k + 1])
        def rw(i):
            pltpu.make_async_copy(rows.at[i], za.at[pl.ds(zoff + row0, _WCH)],
                                  ssems.at[i]).wait()
        for i in range(4):
            pltpu.async_copy(rows.at[i], zb.at[pl.ds(zoff + row0, _WCH)],
                             ssems.at[i])
        def pf(p, _):
            w0 = 2 * p
            rw(0)
            load(w0, 0)
            rw(2)
            load(w0 + 1, 2)
            _gw(0)
            rw(1)
            compute_store(w0, 0)
            _gw(2)
            rw(3)
            compute_store(w0 + 1, 2)
            return 0
        lax.fori_loop(0, _NWCH // 2, pf, 0)
        for i in range(4):
            rw(i)

    def _writeback(last, zdst):
        def load(w, k):
            hb = row0 + w * _WCH
            pltpu.async_copy(acc.at[pl.ds(hb, _WCH)], rows.at[k],
                             gsems.at[k])
            _sum_io(hb, rows.at[k + 1], gsems.at[k + 1], write=False)
        def compute_store(w, k):
            hb = row0 + w * _WCH
            a = rows.at[k]
            b = rows.at[k + 1]
            def gf(g, _):
                dvec = dinv[pl.ds(w * _WCH + g * 16, 16)]
                for r16 in range(16):
                    r = g * 16 + r16
                    di = dvec[r16]
                    s0 = b[r, 0:16] + a[r, 0:16] * di
                    s1 = b[r, 16:32] + a[r, 16:32] * di
                    if last:
                        b[r, 0:16] = s0 * 0.25
                        b[r, 16:32] = s1 * 0.25
                    else:
                        b[r, 0:16] = s0
                        b[r, 16:32] = s1
                        d2 = di * di
                        a[r, 0:16] = a[r, 0:16] * d2
                        a[r, 16:32] = a[r, 16:32] * d2
                return 0
            lax.fori_loop(0, _WCH // 16, gf, 0)
            _sum_io(hb, b, ssems.at[k], write=True)
            if not last:
                pltpu.async_copy(a, zdst.at[pl.ds(zoff + hb, _WCH)],
                                 ssems.at[k + 1])
        def rw(i):
            pltpu.make_async_copy(rows.at[i], za.at[pl.ds(zoff + row0, _WCH)],
                                  ssems.at[i]).wait()
        def entry_waits(k):
            rw(k)
            if not last:
                rw(k + 1)
        dead = za if zdst is zb else zb
        primed = (0, 2) if last else (0, 1, 2, 3)
        for i in primed:
            pltpu.async_copy(rows.at[i], dead.at[pl.ds(zoff + row0, _WCH)],
                             ssems.at[i])
        def pf(p, _):
            w0 = 2 * p
            entry_waits(0)
            load(w0, 0)
            entry_waits(2)
            load(w0 + 1, 2)
            _gw(0)
            _gw(1)
            compute_store(w0, 0)
            _gw(2)
            _gw(3)
            compute_store(w0 + 1, 2)
            return 0
        lax.fori_loop(0, _NWCH // 2, pf, 0)
        for k in (0, 2):
            entry_waits(k)

    for k in range(_CHUNK // 16):
        ones1[pl.ds(k * 16, 16)] = f1
    def zf(g, _):
        dinv[pl.ds(g * 16, 16)] = f0
        return 0
    lax.fori_loop(0, _RPT // 16, zf, 0)
    pltpu.sync_copy(dinv, dacc.at[pl.ds(row0, _RPT)])
    plsc.subcore_barrier()
    _edge_pass(None)
    plsc.subcore_barrier()
    _dinv_phase()
    _z0_phase()

    zsrc = za
    for l in range(3):
        _clear_acc_slice()
        plsc.subcore_barrier()
        _edge_pass(zsrc)
        plsc.subcore_barrier()
        zdst = zb if zsrc is za else za
        _writeback(last=(l == 2), zdst=zdst)
        zsrc = zdst


@functools.cache
def _build_lgcn():
  return functools.partial(
    pl.kernel,
    out_type=(
        jax.ShapeDtypeStruct((_NUM_USERS, _D), jnp.float32),
        jax.ShapeDtypeStruct((_NUM_ITEMS, _D), jnp.float32),
        jax.ShapeDtypeStruct((_NPAD - _N, _D), jnp.float32),
        jax.ShapeDtypeStruct((2 * _NPAD, _HALF), jnp.float32),
        jax.ShapeDtypeStruct((2 * _NPAD, _HALF), jnp.float32),
    ),
    mesh=plsc.VectorSubcoreMesh(core_axis_name="c", subcore_axis_name="s"),
    compiler_params=pltpu.CompilerParams(use_tc_tiling_on_sc=False),
    scratch_types=[
        pltpu.VMEM_SHARED((_NPAD, _HALF), jnp.float32),
        pltpu.VMEM_SHARED((_NPAD,), jnp.float32),
        pltpu.VMEM((_CHUNK,), jnp.float32),
        pltpu.VMEM((2, _BLK, _CHUNK), jnp.int32),
        pltpu.VMEM((2, _BLK, _CHUNK), jnp.int32),
        pltpu.VMEM((_NBUF, _CHUNK, _HALF), jnp.float32),
        pltpu.VMEM((_RPT,), jnp.float32),
        pltpu.SemaphoreType.DMA((_NBUF,)),
        pltpu.SemaphoreType.DMA((_NBUF,)),
        pltpu.SemaphoreType.DMA((2,)),
    ],
  )(_lgcn_body)


def kernel(user_table, item_table, edge_index):
    nblk_tot = _NCHROWS // _BLK
    src3 = edge_index[0].reshape(nblk_tot, _BLK, _CHUNK)
    dst2 = edge_index[1].reshape(nblk_tot, _BLK, _CHUNK)
    user_final, item_final, _, _, _ = _build_lgcn()(user_table, item_table, src3, dst2)
    return user_final, item_final
